# Initial kernel scaffold; baseline (speedup 1.0000x reference)
import jax
import jax.numpy as jnp
from jax.experimental import pallas as pl


def kernel(edge_feats, node_feats, edge_index, rev_index, W, b):
    raise NotImplementedError("write your pallas kernel here")


# speedup vs baseline: 2.1774x; 2.1774x over previous
import functools

import jax
import jax.numpy as jnp
from jax import lax
from jax.experimental import pallas as pl
from jax.experimental.pallas import tpu as pltpu
from jax.experimental.pallas import tpu_sc as plsc

E = 320000
V = 10000
D = 128

NC = 2
NS = 16
NW = NC * NS

CHUNK = 128
NCHUNKS = E // CHUNK
VP = 10240
VSLAB = VP // NS
VSTEP = 128
D16 = D // 16


def _matmul_body(x_ref, w_ref, o_ref):
    x = jnp.maximum(x_ref[...], 0.0)
    o_ref[...] = lax.dot_general(
        x, w_ref[...], (((1,), (1,)), ((), ())),
        preferred_element_type=jnp.float32)


def _relu_matmul(edge_feats, W):
    TM = 1280
    return pl.pallas_call(
        _matmul_body,
        grid=(E // TM,),
        in_specs=[
            pl.BlockSpec((TM, D), lambda i: (i, 0)),
            pl.BlockSpec((D, D), lambda i: (0, 0)),
        ],
        out_specs=pl.BlockSpec((TM, D), lambda i: (i, 0)),
        out_shape=jax.ShapeDtypeStruct((E, D), jnp.float32),
    )(edge_feats, W)


_MESH = plsc.VectorSubcoreMesh(core_axis_name="c", subcore_axis_name="s")


@functools.partial(
    pl.kernel,
    out_type=jax.ShapeDtypeStruct((NC, VP, D), jnp.float32),
    mesh=_MESH,
    scratch_types=[
        pltpu.VMEM_SHARED((VP, D), jnp.float32),
        pltpu.VMEM((CHUNK, D), jnp.float32),
        pltpu.VMEM((CHUNK,), jnp.int32),
    ],
)
def _scatter_kernel(p_hbm, dest_hbm, zero_hbm, part_hbm, accum_s, rows_v, idx_v):
    cid = lax.axis_index("c")
    sid = lax.axis_index("s")
    wid = sid * NC + cid

    slab = sid * VSLAB
    pltpu.sync_copy(zero_hbm.at[pl.ds(0, VSLAB), :],
                    accum_s.at[pl.ds(slab, VSLAB), :])
    plsc.subcore_barrier()

    nchunks = 78 + jnp.where(wid < NCHUNKS - 78 * NW, 1, 0)

    def body(k, _):
        off = (wid + k * NW) * CHUNK
        pltpu.sync_copy(dest_hbm.at[pl.ds(off, CHUNK)], idx_v)
        pltpu.sync_copy(p_hbm.at[pl.ds(off, CHUNK), :], rows_v)
        pltpu.sync_copy(rows_v, accum_s.at[idx_v], add=True)
        return 0

    lax.fori_loop(0, nchunks, body, 0)
    plsc.subcore_barrier()

    pltpu.sync_copy(accum_s.at[pl.ds(slab, VSLAB), :],
                    part_hbm.at[cid, pl.ds(slab, VSLAB), :])


@functools.partial(
    pl.kernel,
    out_type=jax.ShapeDtypeStruct((E, D), jnp.float32),
    mesh=_MESH,
    scratch_types=[
        pltpu.VMEM_SHARED((VP, D), jnp.float32),
        pltpu.VMEM((D,), jnp.float32),
        pltpu.VMEM((CHUNK, D), jnp.float32),
        pltpu.VMEM((CHUNK, D), jnp.float32),
        pltpu.VMEM((CHUNK,), jnp.int32),
        pltpu.VMEM((CHUNK,), jnp.int32),
    ],
)
def _gather_kernel(p_hbm, part_hbm, b_hbm, src_hbm, rev_hbm, out_hbm,
                   nmb_s, b_v, a_v, g_v, sidx_v, ridx_v):
    t0_v, t1_v = a_v, g_v
    cid = lax.axis_index("c")
    sid = lax.axis_index("s")
    wid = sid * NC + cid

    pltpu.sync_copy(b_hbm, b_v)

    def stage(m, _):
        r0 = sid * VSLAB + m * VSTEP
        pltpu.sync_copy(part_hbm.at[0, pl.ds(r0, VSTEP), :], t0_v)
        pltpu.sync_copy(part_hbm.at[1, pl.ds(r0, VSTEP), :], t1_v)

        def row(r, _):
            for j in range(D16):
                sl = pl.ds(j * 16, 16)
                t0_v[r, sl] = t0_v[r, sl] + t1_v[r, sl] + b_v[sl]
            return 0

        lax.fori_loop(0, VSTEP, row, 0)
        pltpu.sync_copy(t0_v, nmb_s.at[pl.ds(r0, VSTEP), :])
        return 0

    lax.fori_loop(0, VSLAB // VSTEP, stage, 0)
    plsc.subcore_barrier()

    nchunks = 78 + jnp.where(wid < NCHUNKS - 78 * NW, 1, 0)

    def body(k, _):
        off = (wid + k * NW) * CHUNK
        pltpu.sync_copy(src_hbm.at[pl.ds(off, CHUNK)], sidx_v)
        pltpu.sync_copy(rev_hbm.at[pl.ds(off, CHUNK)], ridx_v)
        pltpu.sync_copy(nmb_s.at[sidx_v], a_v)
        pltpu.sync_copy(p_hbm.at[ridx_v], g_v)

        def row(r, _):
            for j in range(D16):
                sl = pl.ds(j * 16, 16)
                a_v[r, sl] = a_v[r, sl] - g_v[r, sl]
            return 0

        lax.fori_loop(0, CHUNK, row, 0)
        pltpu.sync_copy(a_v, out_hbm.at[pl.ds(off, CHUNK), :])
        return 0

    lax.fori_loop(0, nchunks, body, 0)


def kernel(edge_feats, node_feats, edge_index, rev_index, W, b):
    del node_feats
    src = edge_index[0]
    dest = edge_index[1]
    p = _relu_matmul(edge_feats, W)
    zeros = jnp.zeros((VSLAB, D), jnp.float32)
    partials = _scatter_kernel(p, dest, zeros)
    return _gather_kernel(p, partials, b, src, rev_index)
